# baseline (speedup 1.0000x reference)
import jax
import jax.numpy as jnp
from jax.experimental import pallas as pl
from jax.experimental.pallas import tpu as pltpu

N = 500
IN_FEAT = 512
OUT_FEAT = 256
HEADS = 3
K_PROP = 10
ALPHA = 0.1


def _fused_kernel(a_hbm_ref, x_ref, w_hbm_ref, asrc_ref, adst_ref,
                  wfc_hbm_ref, bfc_ref, out_ref, a_vmem, w_vmem, wfc_vmem,
                  a_sem, w_sem, wfc_sem):
    a_copy = pltpu.make_async_copy(a_hbm_ref, a_vmem, a_sem)
    a_copy.start()
    w_copy = pltpu.make_async_copy(w_hbm_ref, w_vmem, w_sem)
    w_copy.start()
    wfc_copy = pltpu.make_async_copy(wfc_hbm_ref, wfc_vmem, wfc_sem)
    wfc_copy.start()

    x = x_ref[...]

    denom = jnp.maximum(jnp.sum(jnp.abs(x), axis=0, keepdims=True), 1e-12)
    xn = x / denom

    w_copy.wait()
    Wh = jnp.dot(xn, w_vmem[...], preferred_element_type=jnp.float32)

    a_copy.wait()
    A = a_vmem[...]
    mask = A > 0.0

    heads = []
    for hd in range(HEADS):
        Whh = Wh[:, hd * OUT_FEAT:(hd + 1) * OUT_FEAT]
        es = jnp.sum(Whh * asrc_ref[hd, :][None, :], axis=1)
        ed = jnp.sum(Whh * adst_ref[hd, :][None, :], axis=1)
        e = es[:, None] + ed[None, :]
        e = jnp.where(e >= 0.0, e, 0.2 * e)
        e = jnp.where(mask, e, jnp.float32(-1e9))
        e = e - jnp.max(e, axis=1, keepdims=True)
        p = jnp.exp(e)
        hh = jnp.dot(p, Whh, preferred_element_type=jnp.float32)
        hh = hh / jnp.sum(p, axis=1, keepdims=True)
        hh = jnp.where(hh > 0.0, hh, jnp.exp(jnp.minimum(hh, 0.0)) - 1.0)
        heads.append(hh)
    h0 = jnp.concatenate(heads, axis=1)

    deg = jnp.sum(A, axis=1)
    d_inv_sqrt = jnp.where(deg > 0.0, jax.lax.rsqrt(deg), 0.0)
    A_hat = A * d_inv_sqrt[:, None] * d_inv_sqrt[None, :]

    beta = 1.0 - ALPHA
    Ab = A_hat.astype(jnp.bfloat16)
    A2b = ((beta * beta) * jnp.dot(Ab, Ab, preferred_element_type=jnp.float32)
           ).astype(jnp.bfloat16)
    q = (ALPHA * beta) * jnp.dot(A_hat, h0,
                                 preferred_element_type=jnp.float32) \
        + ALPHA * h0
    wfc_copy.wait()
    wfc = wfc_vmem[...].reshape(2, N, HEADS * OUT_FEAT)

    h = h0
    for _ in range(K_PROP // 2):
        h = jnp.dot(A2b, h.astype(jnp.bfloat16),
                    preferred_element_type=jnp.float32) + q

    prod0 = wfc[0] * h
    prod1 = wfc[1] * h
    ones_col = jnp.ones((HEADS * OUT_FEAT, 1), jnp.float32)
    s0 = jnp.dot(prod0, ones_col, preferred_element_type=jnp.float32)
    s1 = jnp.dot(prod1, ones_col, preferred_element_type=jnp.float32)
    o0 = jnp.sum(s0, keepdims=True)
    o1 = jnp.sum(s1, keepdims=True)
    out_ref[...] = jnp.concatenate([o0, o1], axis=1) + bfc_ref[...][None, :]


def kernel(A, x, W, a_src, a_dst, w_fc, b_fc):
    out = pl.pallas_call(
        _fused_kernel,
        in_specs=[
            pl.BlockSpec(memory_space=pltpu.MemorySpace.HBM),
            pl.BlockSpec(memory_space=pltpu.MemorySpace.VMEM),
            pl.BlockSpec(memory_space=pltpu.MemorySpace.HBM),
            pl.BlockSpec(memory_space=pltpu.MemorySpace.VMEM),
            pl.BlockSpec(memory_space=pltpu.MemorySpace.VMEM),
            pl.BlockSpec(memory_space=pltpu.MemorySpace.HBM),
            pl.BlockSpec(memory_space=pltpu.MemorySpace.VMEM),
        ],
        out_shape=jax.ShapeDtypeStruct((1, 2), jnp.float32),
        scratch_shapes=[
            pltpu.VMEM((N, N), jnp.float32),
            pltpu.VMEM((IN_FEAT, HEADS * OUT_FEAT), jnp.float32),
            pltpu.VMEM((2, HEADS * OUT_FEAT * N), jnp.float32),
            pltpu.SemaphoreType.DMA,
            pltpu.SemaphoreType.DMA,
            pltpu.SemaphoreType.DMA,
        ],
    )(A, x, W, a_src, a_dst, w_fc, b_fc)
    return out[0]


# speedup vs baseline: 1.0801x; 1.0801x over previous
import jax
import jax.numpy as jnp
from jax.experimental import pallas as pl
from jax.experimental.pallas import tpu as pltpu

N = 500
IN_FEAT = 512
OUT_FEAT = 256
HEADS = 3
K_PROP = 10
ALPHA = 0.1


def _fused_kernel(a_hbm_ref, x_ref, w_ref, asrc_ref, adst_ref, wfc_hbm_ref,
                  bfc_ref, out_ref, a_vmem, wfc_vmem, a_sem, wfc_sem):
    a_copy = pltpu.make_async_copy(a_hbm_ref, a_vmem, a_sem)
    a_copy.start()
    wfc_copy = pltpu.make_async_copy(wfc_hbm_ref, wfc_vmem, wfc_sem)
    wfc_copy.start()

    x = x_ref[...]

    denom = jnp.maximum(jnp.sum(jnp.abs(x), axis=0, keepdims=True), 1e-12)
    xn = x / denom

    Wh = jnp.dot(xn, w_ref[...], preferred_element_type=jnp.float32)

    a_copy.wait()
    A = a_vmem[...]
    mask = A > 0.0

    heads = []
    for hd in range(HEADS):
        Whh = Wh[:, hd * OUT_FEAT:(hd + 1) * OUT_FEAT]
        es = jnp.sum(Whh * asrc_ref[hd, :][None, :], axis=1)
        ed = jnp.sum(Whh * adst_ref[hd, :][None, :], axis=1)
        e = es[:, None] + ed[None, :]
        e = jnp.where(e >= 0.0, e, 0.2 * e)
        e = jnp.where(mask, e, jnp.float32(-1e9))
        e = e - jnp.max(e, axis=1, keepdims=True)
        p = jnp.exp(e)
        hh = jnp.dot(p, Whh, preferred_element_type=jnp.float32)
        hh = hh / jnp.sum(p, axis=1, keepdims=True)
        hh = jnp.where(hh > 0.0, hh, jnp.exp(jnp.minimum(hh, 0.0)) - 1.0)
        heads.append(hh)
    h0 = jnp.concatenate(heads, axis=1)

    deg = jnp.sum(A, axis=1)
    d_inv_sqrt = jnp.where(deg > 0.0, jax.lax.rsqrt(deg), 0.0)
    A_hat = A * d_inv_sqrt[:, None] * d_inv_sqrt[None, :]

    beta = 1.0 - ALPHA
    Ab = A_hat.astype(jnp.bfloat16)
    A2b = ((beta * beta) * jnp.dot(Ab, Ab, preferred_element_type=jnp.float32)
           ).astype(jnp.bfloat16)
    q = (ALPHA * beta) * jnp.dot(A_hat, h0,
                                 preferred_element_type=jnp.float32) \
        + ALPHA * h0
    wfc_copy.wait()
    wfc = wfc_vmem[...].reshape(2, N, HEADS * OUT_FEAT)

    h = h0
    for _ in range(K_PROP // 2):
        h = jnp.dot(A2b, h.astype(jnp.bfloat16),
                    preferred_element_type=jnp.float32) + q

    prod0 = wfc[0] * h
    prod1 = wfc[1] * h
    ones_col = jnp.ones((HEADS * OUT_FEAT, 1), jnp.float32)
    s0 = jnp.dot(prod0, ones_col, preferred_element_type=jnp.float32)
    s1 = jnp.dot(prod1, ones_col, preferred_element_type=jnp.float32)
    o0 = jnp.sum(s0, keepdims=True)
    o1 = jnp.sum(s1, keepdims=True)
    out_ref[...] = jnp.concatenate([o0, o1], axis=1) + bfc_ref[...][None, :]


def kernel(A, x, W, a_src, a_dst, w_fc, b_fc):
    out = pl.pallas_call(
        _fused_kernel,
        in_specs=[
            pl.BlockSpec(memory_space=pltpu.MemorySpace.HBM),
            pl.BlockSpec(memory_space=pltpu.MemorySpace.VMEM),
            pl.BlockSpec(memory_space=pltpu.MemorySpace.VMEM),
            pl.BlockSpec(memory_space=pltpu.MemorySpace.VMEM),
            pl.BlockSpec(memory_space=pltpu.MemorySpace.VMEM),
            pl.BlockSpec(memory_space=pltpu.MemorySpace.HBM),
            pl.BlockSpec(memory_space=pltpu.MemorySpace.VMEM),
        ],
        out_shape=jax.ShapeDtypeStruct((1, 2), jnp.float32),
        scratch_shapes=[
            pltpu.VMEM((N, N), jnp.float32),
            pltpu.VMEM((2, HEADS * OUT_FEAT * N), jnp.float32),
            pltpu.SemaphoreType.DMA,
            pltpu.SemaphoreType.DMA,
        ],
    )(A, x, W, a_src, a_dst, w_fc, b_fc)
    return out[0]


# speedup vs baseline: 1.1165x; 1.0337x over previous
import jax
import jax.numpy as jnp
from jax.experimental import pallas as pl
from jax.experimental.pallas import tpu as pltpu

N = 500
IN_FEAT = 512
OUT_FEAT = 256
HEADS = 3
K_PROP = 10
ALPHA = 0.1


def _fused_kernel(a_ref, x_ref, w_ref, asrc_ref, adst_ref, wfc_hbm_ref,
                  bfc_ref, out_ref, wfc_vmem, wfc_sem):
    wfc_copy = pltpu.make_async_copy(wfc_hbm_ref, wfc_vmem, wfc_sem)
    wfc_copy.start()

    A = a_ref[...]
    x = x_ref[...]

    denom = jnp.maximum(jnp.sum(jnp.abs(x), axis=0, keepdims=True), 1e-12)
    xn = x / denom

    Wh = jnp.dot(xn, w_ref[...], preferred_element_type=jnp.float32)

    mask = A > 0.0

    heads = []
    for hd in range(HEADS):
        Whh = Wh[:, hd * OUT_FEAT:(hd + 1) * OUT_FEAT]
        es = jnp.sum(Whh * asrc_ref[hd, :][None, :], axis=1)
        ed = jnp.sum(Whh * adst_ref[hd, :][None, :], axis=1)
        e = es[:, None] + ed[None, :]
        e = jnp.where(e >= 0.0, e, 0.2 * e)
        e = jnp.where(mask, e, jnp.float32(-1e9))
        e = e - jnp.max(e, axis=1, keepdims=True)
        p = jnp.exp(e)
        hh = jnp.dot(p, Whh, preferred_element_type=jnp.float32)
        hh = hh / jnp.sum(p, axis=1, keepdims=True)
        hh = jnp.where(hh > 0.0, hh, jnp.exp(jnp.minimum(hh, 0.0)) - 1.0)
        heads.append(hh)
    h0 = jnp.concatenate(heads, axis=1)

    deg = jnp.sum(A, axis=1)
    d_inv_sqrt = jnp.where(deg > 0.0, jax.lax.rsqrt(deg), 0.0)
    A_hat = A * d_inv_sqrt[:, None] * d_inv_sqrt[None, :]

    beta = 1.0 - ALPHA
    Ab = A_hat.astype(jnp.bfloat16)
    A2b = ((beta * beta) * jnp.dot(Ab, Ab, preferred_element_type=jnp.float32)
           ).astype(jnp.bfloat16)
    q = (ALPHA * beta) * jnp.dot(A_hat, h0,
                                 preferred_element_type=jnp.float32) \
        + ALPHA * h0
    wfc_copy.wait()
    wfc = wfc_vmem[...].reshape(2, N, HEADS * OUT_FEAT)

    h = h0
    for _ in range(K_PROP // 2):
        h = jnp.dot(A2b, h.astype(jnp.bfloat16),
                    preferred_element_type=jnp.float32) + q

    prod0 = wfc[0] * h
    prod1 = wfc[1] * h
    ones_col = jnp.ones((HEADS * OUT_FEAT, 1), jnp.float32)
    s0 = jnp.dot(prod0, ones_col, preferred_element_type=jnp.float32)
    s1 = jnp.dot(prod1, ones_col, preferred_element_type=jnp.float32)
    o0 = jnp.sum(s0, keepdims=True)
    o1 = jnp.sum(s1, keepdims=True)
    out_ref[...] = jnp.concatenate([o0, o1], axis=1) + bfc_ref[...][None, :]


def kernel(A, x, W, a_src, a_dst, w_fc, b_fc):
    out = pl.pallas_call(
        _fused_kernel,
        in_specs=[
            pl.BlockSpec(memory_space=pltpu.MemorySpace.VMEM),
            pl.BlockSpec(memory_space=pltpu.MemorySpace.VMEM),
            pl.BlockSpec(memory_space=pltpu.MemorySpace.VMEM),
            pl.BlockSpec(memory_space=pltpu.MemorySpace.VMEM),
            pl.BlockSpec(memory_space=pltpu.MemorySpace.VMEM),
            pl.BlockSpec(memory_space=pltpu.MemorySpace.HBM),
            pl.BlockSpec(memory_space=pltpu.MemorySpace.VMEM),
        ],
        out_shape=jax.ShapeDtypeStruct((1, 2), jnp.float32),
        scratch_shapes=[
            pltpu.VMEM((2, HEADS * OUT_FEAT * N), jnp.float32),
            pltpu.SemaphoreType.DMA,
        ],
    )(A, x, W, a_src, a_dst, w_fc, b_fc)
    return out[0]
